# baseline (speedup 1.0000x reference)
import functools

import jax
import jax.numpy as jnp
from jax import lax
from jax.experimental import pallas as pl
from jax.experimental.pallas import tpu as pltpu
from jax.experimental.pallas import tpu_sc as plsc

NC = 2
NS = 16
NW = NC * NS

B, C, H, W = 4, 96, 224, 224
PLANE = H * W
K = 8
CW = PLANE // K
NPLANES = B * C
PPW = NPLANES // NW


def _shuffle_body(x_rows, cids, out, idx_v, buf0, buf1, gsem, ssem):
    wid = lax.axis_index("s") * NC + lax.axis_index("c")
    base = wid * PPW
    pltpu.sync_copy(cids.at[wid], idx_v)

    bufs = (buf0, buf1)
    gathers = [None] * PPW
    writes = [None] * PPW
    for j in range(PPW):
        if j >= 2:
            writes[j - 2].wait()
        gathers[j] = pltpu.async_copy(
            x_rows.at[idx_v.at[j]], bufs[j % 2], gsem)
        if j >= 1:
            gathers[j - 1].wait()
            writes[j - 1] = pltpu.async_copy(
                bufs[(j - 1) % 2], out.at[base + j - 1], ssem)
    gathers[PPW - 1].wait()
    writes[PPW - 1] = pltpu.async_copy(
        bufs[(PPW - 1) % 2], out.at[base + PPW - 1], ssem)
    writes[PPW - 2].wait()
    writes[PPW - 1].wait()


@functools.partial(jax.jit, static_argnames=())
def _shuffle(x_rows, cids):
    run = pl.kernel(
        _shuffle_body,
        out_type=jax.ShapeDtypeStruct((NPLANES, K, CW), jnp.float32),
        mesh=plsc.VectorSubcoreMesh(core_axis_name="c", subcore_axis_name="s"),
        scratch_types=[
            pltpu.VMEM((PPW, K), jnp.int32),
            pltpu.VMEM((K, CW), jnp.float32),
            pltpu.VMEM((K, CW), jnp.float32),
            pltpu.SemaphoreType.DMA,
            pltpu.SemaphoreType.DMA,
        ],
    )
    return run(x_rows, cids)


def kernel(x, forward_shuffle_idx):
    src_plane = (jnp.arange(B, dtype=jnp.int32)[:, None] * C
                 + forward_shuffle_idx[None, :]).reshape(-1)
    cids = (src_plane[:, None] * K
            + jnp.arange(K, dtype=jnp.int32)[None, :])
    cids = cids.reshape(NW, PPW, K)
    x_rows = x.reshape(NPLANES * K, CW)
    out = _shuffle(x_rows, cids)
    return (out.reshape(B, C, H, W), 0)


# speedup vs baseline: 3.2694x; 3.2694x over previous
import functools

import jax
import jax.numpy as jnp
from jax import lax
from jax.experimental import pallas as pl
from jax.experimental.pallas import tpu as pltpu
from jax.experimental.pallas import tpu_sc as plsc

NC = 2
NS = 16
NW = NC * NS

B, C, H, W = 4, 96, 224, 224
NPLANES = B * C
PPW = NPLANES // NW
LANE = 16


def _shuffle_body(x3, srcs, out, idx_v, buf0, buf1, gsem, ssem):
    wid = lax.axis_index("s") * NC + lax.axis_index("c")
    base = wid * PPW
    pltpu.sync_copy(srcs.at[wid], idx_v)
    ids = idx_v[...]

    def src_scalar(j):
        return ids[j]

    bufs = (buf0, buf1)
    gathers = [None] * PPW
    writes = [None] * PPW
    for j in range(PPW):
        if j >= 2:
            writes[j - 2].wait()
        gathers[j] = pltpu.async_copy(x3.at[src_scalar(j)], bufs[j % 2], gsem)
        if j >= 1:
            gathers[j - 1].wait()
            writes[j - 1] = pltpu.async_copy(
                bufs[(j - 1) % 2], out.at[base + j - 1], ssem)
    gathers[PPW - 1].wait()
    writes[PPW - 1] = pltpu.async_copy(
        bufs[(PPW - 1) % 2], out.at[base + PPW - 1], ssem)
    writes[PPW - 2].wait()
    writes[PPW - 1].wait()


@jax.jit
def _shuffle(x3, srcs):
    run = pl.kernel(
        _shuffle_body,
        out_type=jax.ShapeDtypeStruct((NPLANES, H, W), jnp.float32),
        mesh=plsc.VectorSubcoreMesh(core_axis_name="c", subcore_axis_name="s"),
        scratch_types=[
            pltpu.VMEM((LANE,), jnp.int32),
            pltpu.VMEM((H, W), jnp.float32),
            pltpu.VMEM((H, W), jnp.float32),
            pltpu.SemaphoreType.DMA,
            pltpu.SemaphoreType.DMA,
        ],
    )
    return run(x3, srcs)


def kernel(x, forward_shuffle_idx):
    src_plane = (jnp.arange(B, dtype=jnp.int32)[:, None] * C
                 + forward_shuffle_idx[None, :]).reshape(NW, PPW)
    srcs = jnp.pad(src_plane, ((0, 0), (0, LANE - PPW)))
    out = _shuffle(x.reshape(NPLANES, H, W), srcs)
    return (out.reshape(B, C, H, W), 0)
